# Initial kernel scaffold; baseline (speedup 1.0000x reference)
import jax
import jax.numpy as jnp
from jax.experimental import pallas as pl


def kernel(X, pair_v, pair_e, W1, b1, ae1, W2, b2, ae2):
    raise NotImplementedError("write your pallas kernel here")


# speedup vs baseline: 17.9914x; 17.9914x over previous
import functools

import jax
import jax.numpy as jnp
from jax import lax
from jax.experimental import pallas as pl
from jax.experimental.pallas import tpu as pltpu
from jax.experimental.pallas import tpu_sc as plsc

N_V = 10000
N_E = 5000
P = 320000
H = 4
IN_C = 128
HID = 64
CLS = 40
NEG = 0.2

NC, NS = 2, 16
CH = 128
P_PAD = 327680
V_TBL = 10008
E_TBL = 5008
ACC_E = 5024
ACC_V = 10048
ZR = 157


def _seg_pass(D, acc_rows, gather_edge, col_split, tbl_rows):
    mesh = plsc.VectorSubcoreMesh(core_axis_name="c", subcore_axis_name="s")
    per_tile = P_PAD // NS if col_split else P_PAD // (NS * NC)
    n_chunks = per_tile // CH
    rpt = acc_rows // NS

    @functools.partial(
        pl.kernel,
        mesh=mesh,
        compiler_params=pltpu.CompilerParams(use_tc_tiling_on_sc=False),
        out_type=jax.ShapeDtypeStruct((NC * acc_rows, D), jnp.float32),
        scratch_types=[
            pltpu.VMEM((CH,), jnp.int32),
            pltpu.VMEM((CH,), jnp.int32),
            pltpu.VMEM((CH,), jnp.int32),
            pltpu.VMEM((CH, D), jnp.float32),
            pltpu.VMEM((ZR, D), jnp.float32),
            pltpu.VMEM_SHARED((acc_rows, D), jnp.float32),
            pltpu.SemaphoreType.DMA,
        ],
    )
    def k(tbl, pairs, out, idxp_v, idxg_v, idxs_v, rows_v, zbuf, acc, sem):
        c = lax.axis_index("c")
        s = lax.axis_index("s")

        def zrow(i, carry):
            for j in range(D // 16):
                zbuf[i, pl.ds(j * 16, 16)] = jnp.zeros((16,), jnp.float32)
            return carry

        lax.fori_loop(0, ZR, zrow, 0)

        def zacc(kk, carry):
            pltpu.sync_copy(zbuf, acc.at[pl.ds(s * rpt + kk * ZR, ZR)])
            return carry

        lax.fori_loop(0, rpt // ZR, zacc, 0)
        plsc.subcore_barrier()

        if col_split:
            start = s * per_tile
            goff = c * (tbl_rows // 2)
        else:
            start = (s * NC + c) * per_tile
            goff = 0

        def chunk(kk, carry):
            off = start + kk * CH
            pltpu.sync_copy(pairs.at[pl.ds(off, CH)], idxp_v)
            for j in range(CH // 16):
                sl = pl.ds(j * 16, 16)
                v = idxp_v[sl]
                pe = lax.shift_right_logical(v, 14)
                pv = lax.bitwise_and(v, 16383)
                idxg_v[sl] = (pe if gather_edge else pv) + goff
                idxs_v[sl] = pv if gather_edge else pe
            pltpu.async_copy(tbl.at[idxg_v], rows_v, sem).wait()
            pltpu.sync_copy(rows_v, acc.at[idxs_v], add=True)
            return carry

        lax.fori_loop(0, n_chunks, chunk, 0)
        plsc.subcore_barrier()

        def wb(kk, carry):
            r = s * rpt + kk * ZR
            pltpu.sync_copy(acc.at[pl.ds(r, ZR)], out.at[pl.ds(c * acc_rows + r, ZR)])
            return carry

        lax.fori_loop(0, rpt // ZR, wb, 0)

    return k


_pass1 = _seg_pass(144, ACC_E, gather_edge=False, col_split=False, tbl_rows=V_TBL)
_pass2 = _seg_pass(72, ACC_V, gather_edge=True, col_split=True, tbl_rows=2 * E_TBL)
_pass3 = _seg_pass(128, ACC_E, gather_edge=False, col_split=True, tbl_rows=2 * V_TBL)
_pass4 = _seg_pass(48, ACC_V, gather_edge=True, col_split=False, tbl_rows=E_TBL)


def _tc_call(body, out_shapes):
    return pl.pallas_call(body, out_shape=out_shapes)


def _row_mask(x):
    return jnp.where(lax.broadcasted_iota(jnp.int32, (E_TBL, 1), 0) < N_E, x, 0.0)


def _kb(p1, w1, b1, ae1, cnt_ref, t1a_ref, t1b_ref):
    S = p1[0:E_TBL, :] + p1[ACC_E:ACC_E + E_TBL, :]
    cnt = S[:, IN_C]
    cntc = jnp.maximum(cnt, 1.0)
    Xagg = S[:, :IN_C] / cntc[:, None]
    Yall = jnp.dot(Xagg, w1[...], preferred_element_type=jnp.float32) + b1[...][None, :]
    YEs, Es = [], []
    for h in range(H):
        Yh = Yall[:, h * HID:(h + 1) * HID]
        a = jnp.dot(Yh, ae1[h, :], preferred_element_type=jnp.float32)
        sle = jnp.where(a >= 0, a, NEG * a)
        E = jnp.exp(sle - jnp.max(sle))
        YEs.append(Yh * E[:, None])
        Es.append(E[:, None])
    z4 = jnp.zeros((E_TBL, 4), jnp.float32)
    q0 = _row_mask(jnp.concatenate([YEs[0]] + Es + [z4], axis=1))
    q1 = _row_mask(jnp.concatenate([YEs[1], z4, z4], axis=1))
    q2 = _row_mask(jnp.concatenate([YEs[2], z4, z4], axis=1))
    q3 = _row_mask(jnp.concatenate([YEs[3], z4, z4], axis=1))
    t1a_ref[...] = jnp.concatenate([q0, q1], axis=0)
    t1b_ref[...] = jnp.concatenate([q2, q3], axis=0)
    cnt_ref[...] = cnt


def _kd(p2a, p2b, xh_ref):
    qs = [p2a[0:V_TBL, :], p2a[ACC_V:ACC_V + V_TBL, :],
          p2b[0:V_TBL, :], p2b[ACC_V:ACC_V + V_TBL, :]]
    den = qs[0][:, HID:HID + H]
    R = 1.0 / (den + 1e-12)
    o = jnp.concatenate(
        [qs[h][:, :HID] * R[:, h:h + 1] for h in range(H)], axis=1)
    Xh = jnp.where(o > 0, o, jnp.exp(jnp.minimum(o, 0.0)) - 1.0)
    xh_ref[...] = jnp.concatenate([Xh[:, :IN_C], Xh[:, IN_C:]], axis=0)


def _ke(p3, cnt, w2, b2, ae2, t2_ref):
    S2 = jnp.concatenate([p3[0:E_TBL, :], p3[ACC_E:ACC_E + E_TBL, :]], axis=1)
    cntc = jnp.maximum(cnt[...], 1.0)
    X2 = S2 / cntc[:, None]
    Y2 = jnp.dot(X2, w2[...], preferred_element_type=jnp.float32) + b2[...][None, :]
    a = jnp.dot(Y2, ae2[...], preferred_element_type=jnp.float32)
    sle = jnp.where(a >= 0, a, NEG * a)
    E2 = jnp.exp(sle - jnp.max(sle))
    t2_ref[...] = _row_mask(jnp.concatenate(
        [Y2 * E2[:, None], E2[:, None], jnp.zeros((E_TBL, 7), jnp.float32)], axis=1))


def _kf(p4, out_ref):
    acc = p4[0:N_V, :] + p4[ACC_V:ACC_V + N_V, :]
    o = acc[:, :CLS] / (acc[:, CLS:CLS + 1] + 1e-12)
    out_ref[...] = jnp.where(o > 0, o, jnp.exp(jnp.minimum(o, 0.0)) - 1.0)


def kernel(X, pair_v, pair_e, W1, b1, ae1, W2, b2, ae2):
    f32 = jnp.float32
    npad = P_PAD - P
    pv = jnp.concatenate([pair_v, jnp.full((npad,), N_V, jnp.int32)])
    pe = jnp.concatenate([pair_e, jnp.full((npad,), N_E, jnp.int32)])
    pairs = jnp.left_shift(pe, 14) | pv
    X1t = jnp.concatenate(
        [jnp.concatenate([X, jnp.ones((N_V, 1), f32), jnp.zeros((N_V, 15), f32)], axis=1),
         jnp.zeros((V_TBL - N_V, 144), f32)], axis=0)
    W1cat = jnp.transpose(W1, (1, 0, 2)).reshape(IN_C, H * HID)
    b1cat = b1.reshape(H * HID)

    p1 = _pass1(X1t, pairs)
    cnt, T1a, T1b = _tc_call(
        _kb, (jax.ShapeDtypeStruct((E_TBL,), f32),
              jax.ShapeDtypeStruct((2 * E_TBL, 72), f32),
              jax.ShapeDtypeStruct((2 * E_TBL, 72), f32)))(p1, W1cat, b1cat, ae1)
    p2a = _pass2(T1a, pairs)
    p2b = _pass2(T1b, pairs)
    Xh2 = _tc_call(_kd, jax.ShapeDtypeStruct((2 * V_TBL, IN_C), f32))(p2a, p2b)
    p3 = _pass3(Xh2, pairs)
    T2 = _tc_call(_ke, jax.ShapeDtypeStruct((E_TBL, 48), f32))(p3, cnt, W2, b2, ae2)
    p4 = _pass4(T2, pairs)
    out = _tc_call(_kf, jax.ShapeDtypeStruct((N_V, CLS), f32))(p4)
    return out


# speedup vs baseline: 24.3661x; 1.3543x over previous
import functools

import jax
import jax.numpy as jnp
from jax import lax
from jax.experimental import pallas as pl
from jax.experimental.pallas import tpu as pltpu
from jax.experimental.pallas import tpu_sc as plsc

N_V = 10000
N_E = 5000
P = 320000
H = 4
IN_C = 128
HID = 64
CLS = 40
NEG = 0.2

NC, NS = 2, 16
CH = 128
P_PAD = 327680
V_TBL = 10008
E_TBL = 5008
ACC_E = 5024
ACC_V = 10048
ZR = 157


def _seg_pass(D, acc_rows, gather_edge, col_split, tbl_rows):
    mesh = plsc.VectorSubcoreMesh(core_axis_name="c", subcore_axis_name="s")
    per_tile = P_PAD // NS if col_split else P_PAD // (NS * NC)
    n_chunks = per_tile // CH
    rpt = acc_rows // NS

    @functools.partial(
        pl.kernel,
        mesh=mesh,
        compiler_params=pltpu.CompilerParams(use_tc_tiling_on_sc=False),
        out_type=jax.ShapeDtypeStruct((NC * acc_rows, D), jnp.float32),
        scratch_types=[
            pltpu.VMEM((CH,), jnp.int32),
            pltpu.VMEM((CH,), jnp.int32),
            pltpu.VMEM((CH,), jnp.int32),
            pltpu.VMEM((CH,), jnp.int32),
            pltpu.VMEM((CH,), jnp.int32),
            pltpu.VMEM((CH,), jnp.int32),
            pltpu.VMEM((CH, D), jnp.float32),
            pltpu.VMEM((CH, D), jnp.float32),
            pltpu.VMEM((ZR, D), jnp.float32),
            pltpu.VMEM_SHARED((acc_rows, D), jnp.float32),
            pltpu.SemaphoreType.DMA,
            pltpu.SemaphoreType.DMA,
        ],
    )
    def k(tbl, pairs, out, idxp0, idxg0, idxs0, idxp1, idxg1, idxs1,
          rows0, rows1, zbuf, acc, sem0, sem1):
        c = lax.axis_index("c")
        s = lax.axis_index("s")

        def zrow(i, carry):
            for j in range(D // 16):
                zbuf[i, pl.ds(j * 16, 16)] = jnp.zeros((16,), jnp.float32)
            return carry

        lax.fori_loop(0, ZR, zrow, 0)

        def zacc(kk, carry):
            pltpu.sync_copy(zbuf, acc.at[pl.ds(s * rpt + kk * ZR, ZR)])
            return carry

        lax.fori_loop(0, rpt // ZR, zacc, 0)
        plsc.subcore_barrier()

        if col_split:
            start = s * per_tile
            goff = c * (tbl_rows // 2)
        else:
            start = (s * NC + c) * per_tile
            goff = 0

        def prefetch(kk, idxp, idxg, idxs, rows, sem):
            pltpu.sync_copy(pairs.at[pl.ds(start + kk * CH, CH)], idxp)
            for j in range(CH // 16):
                sl = pl.ds(j * 16, 16)
                v = idxp[sl]
                pe = lax.shift_right_logical(v, 14)
                pv = lax.bitwise_and(v, 16383)
                idxg[sl] = (pe if gather_edge else pv) + goff
                idxs[sl] = pv if gather_edge else pe
            pltpu.async_copy(tbl.at[idxg], rows, sem)

        def drain(kk, nxt, idxg, idxs, rows, sem):
            pltpu.make_async_copy(tbl.at[idxg], rows, sem).wait()
            pltpu.sync_copy(rows, acc.at[idxs], add=True)

        prefetch(0, idxp0, idxg0, idxs0, rows0, sem0)
        prefetch(1, idxp1, idxg1, idxs1, rows1, sem1)

        def step(i, carry):
            drain(2 * i, 2 * i + 2, idxg0, idxs0, rows0, sem0)

            @pl.when(2 * i + 2 < n_chunks)
            def _():
                prefetch(2 * i + 2, idxp0, idxg0, idxs0, rows0, sem0)

            drain(2 * i + 1, 2 * i + 3, idxg1, idxs1, rows1, sem1)

            @pl.when(2 * i + 3 < n_chunks)
            def _():
                prefetch(2 * i + 3, idxp1, idxg1, idxs1, rows1, sem1)

            return carry

        lax.fori_loop(0, n_chunks // 2, step, 0)
        plsc.subcore_barrier()

        def wb(kk, carry):
            r = s * rpt + kk * ZR
            pltpu.sync_copy(acc.at[pl.ds(r, ZR)], out.at[pl.ds(c * acc_rows + r, ZR)])
            return carry

        lax.fori_loop(0, rpt // ZR, wb, 0)

    return k


_pass1 = _seg_pass(144, ACC_E, gather_edge=False, col_split=False, tbl_rows=V_TBL)
_pass2 = _seg_pass(72, ACC_V, gather_edge=True, col_split=True, tbl_rows=2 * E_TBL)
_pass3 = _seg_pass(128, ACC_E, gather_edge=False, col_split=True, tbl_rows=2 * V_TBL)
_pass4 = _seg_pass(48, ACC_V, gather_edge=True, col_split=False, tbl_rows=E_TBL)


def _tc_call(body, out_shapes):
    return pl.pallas_call(body, out_shape=out_shapes)


def _row_mask(x):
    return jnp.where(lax.broadcasted_iota(jnp.int32, (E_TBL, 1), 0) < N_E, x, 0.0)


def _kb(p1, w1, b1, ae1, cnt_ref, t1a_ref, t1b_ref):
    S = p1[0:E_TBL, :] + p1[ACC_E:ACC_E + E_TBL, :]
    cnt = S[:, IN_C]
    cntc = jnp.maximum(cnt, 1.0)
    Xagg = S[:, :IN_C] / cntc[:, None]
    Yall = jnp.dot(Xagg, w1[...], preferred_element_type=jnp.float32) + b1[...][None, :]
    YEs, Es = [], []
    for h in range(H):
        Yh = Yall[:, h * HID:(h + 1) * HID]
        a = jnp.dot(Yh, ae1[h, :], preferred_element_type=jnp.float32)
        sle = jnp.where(a >= 0, a, NEG * a)
        E = jnp.exp(sle - jnp.max(sle))
        YEs.append(Yh * E[:, None])
        Es.append(E[:, None])
    z4 = jnp.zeros((E_TBL, 4), jnp.float32)
    q0 = _row_mask(jnp.concatenate([YEs[0]] + Es + [z4], axis=1))
    q1 = _row_mask(jnp.concatenate([YEs[1], z4, z4], axis=1))
    q2 = _row_mask(jnp.concatenate([YEs[2], z4, z4], axis=1))
    q3 = _row_mask(jnp.concatenate([YEs[3], z4, z4], axis=1))
    t1a_ref[...] = jnp.concatenate([q0, q1], axis=0)
    t1b_ref[...] = jnp.concatenate([q2, q3], axis=0)
    cnt_ref[...] = cnt


def _kd(p2a, p2b, xh_ref):
    qs = [p2a[0:V_TBL, :], p2a[ACC_V:ACC_V + V_TBL, :],
          p2b[0:V_TBL, :], p2b[ACC_V:ACC_V + V_TBL, :]]
    den = qs[0][:, HID:HID + H]
    R = 1.0 / (den + 1e-12)
    o = jnp.concatenate(
        [qs[h][:, :HID] * R[:, h:h + 1] for h in range(H)], axis=1)
    Xh = jnp.where(o > 0, o, jnp.exp(jnp.minimum(o, 0.0)) - 1.0)
    xh_ref[...] = jnp.concatenate([Xh[:, :IN_C], Xh[:, IN_C:]], axis=0)


def _ke(p3, cnt, w2, b2, ae2, t2_ref):
    S2 = jnp.concatenate([p3[0:E_TBL, :], p3[ACC_E:ACC_E + E_TBL, :]], axis=1)
    cntc = jnp.maximum(cnt[...], 1.0)
    X2 = S2 / cntc[:, None]
    Y2 = jnp.dot(X2, w2[...], preferred_element_type=jnp.float32) + b2[...][None, :]
    a = jnp.dot(Y2, ae2[...], preferred_element_type=jnp.float32)
    sle = jnp.where(a >= 0, a, NEG * a)
    E2 = jnp.exp(sle - jnp.max(sle))
    t2_ref[...] = _row_mask(jnp.concatenate(
        [Y2 * E2[:, None], E2[:, None], jnp.zeros((E_TBL, 7), jnp.float32)], axis=1))


def _kf(p4, out_ref):
    acc = p4[0:N_V, :] + p4[ACC_V:ACC_V + N_V, :]
    o = acc[:, :CLS] / (acc[:, CLS:CLS + 1] + 1e-12)
    out_ref[...] = jnp.where(o > 0, o, jnp.exp(jnp.minimum(o, 0.0)) - 1.0)


def kernel(X, pair_v, pair_e, W1, b1, ae1, W2, b2, ae2):
    f32 = jnp.float32
    npad = P_PAD - P
    pv = jnp.concatenate([pair_v, jnp.full((npad,), N_V, jnp.int32)])
    pe = jnp.concatenate([pair_e, jnp.full((npad,), N_E, jnp.int32)])
    pairs = jnp.left_shift(pe, 14) | pv
    X1t = jnp.concatenate(
        [jnp.concatenate([X, jnp.ones((N_V, 1), f32), jnp.zeros((N_V, 15), f32)], axis=1),
         jnp.zeros((V_TBL - N_V, 144), f32)], axis=0)
    W1cat = jnp.transpose(W1, (1, 0, 2)).reshape(IN_C, H * HID)
    b1cat = b1.reshape(H * HID)

    p1 = _pass1(X1t, pairs)
    cnt, T1a, T1b = _tc_call(
        _kb, (jax.ShapeDtypeStruct((E_TBL,), f32),
              jax.ShapeDtypeStruct((2 * E_TBL, 72), f32),
              jax.ShapeDtypeStruct((2 * E_TBL, 72), f32)))(p1, W1cat, b1cat, ae1)
    p2a = _pass2(T1a, pairs)
    p2b = _pass2(T1b, pairs)
    Xh2 = _tc_call(_kd, jax.ShapeDtypeStruct((2 * V_TBL, IN_C), f32))(p2a, p2b)
    p3 = _pass3(Xh2, pairs)
    T2 = _tc_call(_ke, jax.ShapeDtypeStruct((E_TBL, 48), f32))(p3, cnt, W2, b2, ae2)
    p4 = _pass4(T2, pairs)
    out = _tc_call(_kf, jax.ShapeDtypeStruct((N_V, CLS), f32))(p4)
    return out


# speedup vs baseline: 24.9203x; 1.0227x over previous
import functools

import jax
import jax.numpy as jnp
from jax import lax
from jax.experimental import pallas as pl
from jax.experimental.pallas import tpu as pltpu
from jax.experimental.pallas import tpu_sc as plsc

N_V = 10000
N_E = 5000
P = 320000
H = 4
IN_C = 128
HID = 64
CLS = 40
NEG = 0.2

NC, NS = 2, 16
CH = 128
P_PAD = 327680
V_TBL = 10008
E_TBL = 5008
ACC_E = 5024
ACC_V = 10048
ZR = 157


def _seg_pass(D, acc_rows, gather_edge, col_split, tbl_rows, NB):
    mesh = plsc.VectorSubcoreMesh(core_axis_name="c", subcore_axis_name="s")
    per_tile = P_PAD // NS if col_split else P_PAD // (NS * NC)
    n_chunks = per_tile // CH
    rpt = acc_rows // NS

    @functools.partial(
        pl.kernel,
        mesh=mesh,
        compiler_params=pltpu.CompilerParams(use_tc_tiling_on_sc=False),
        out_type=jax.ShapeDtypeStruct((NC * acc_rows, D), jnp.float32),
        scratch_types=(
            [pltpu.VMEM((CH,), jnp.int32)] * (3 * NB)
            + [pltpu.VMEM((CH, D), jnp.float32)] * NB
            + [pltpu.VMEM((ZR, D), jnp.float32),
               pltpu.VMEM_SHARED((acc_rows, D), jnp.float32)]
            + [pltpu.SemaphoreType.DMA] * NB
        ),
    )
    def k(tbl, pairs, out, *scr):
        idxps = scr[0:NB]
        idxgs = scr[NB:2 * NB]
        idxss = scr[2 * NB:3 * NB]
        rowss = scr[3 * NB:4 * NB]
        zbuf = scr[4 * NB]
        acc = scr[4 * NB + 1]
        sems = scr[4 * NB + 2:4 * NB + 2 + NB]
        c = lax.axis_index("c")
        s = lax.axis_index("s")

        def zrow(i, carry):
            for j in range(D // 16):
                zbuf[i, pl.ds(j * 16, 16)] = jnp.zeros((16,), jnp.float32)
            return carry

        lax.fori_loop(0, ZR, zrow, 0)

        def zacc(kk, carry):
            pltpu.sync_copy(zbuf, acc.at[pl.ds(s * rpt + kk * ZR, ZR)])
            return carry

        lax.fori_loop(0, rpt // ZR, zacc, 0)
        plsc.subcore_barrier()

        if col_split:
            start = s * per_tile
            goff = c * (tbl_rows // 2)
        else:
            start = (s * NC + c) * per_tile
            goff = 0

        def prefetch(kk, b):
            pltpu.sync_copy(pairs.at[pl.ds(start + kk * CH, CH)], idxps[b])
            for j in range(CH // 16):
                sl = pl.ds(j * 16, 16)
                v = idxps[b][sl]
                pe = lax.shift_right_logical(v, 14)
                pv = lax.bitwise_and(v, 16383)
                idxgs[b][sl] = (pe if gather_edge else pv) + goff
                idxss[b][sl] = pv if gather_edge else pe
            pltpu.async_copy(tbl.at[idxgs[b]], rowss[b], sems[b])

        def drain(b):
            pltpu.make_async_copy(tbl.at[idxgs[b]], rowss[b], sems[b]).wait()
            pltpu.sync_copy(rowss[b], acc.at[idxss[b]], add=True)

        for b in range(NB):
            prefetch(b, b)

        def step(i, carry):
            for b in range(NB):
                drain(b)
                nxt = NB * i + b + NB

                @pl.when(nxt < n_chunks)
                def _(nxt=nxt, b=b):
                    prefetch(nxt, b)

            return carry

        lax.fori_loop(0, n_chunks // NB, step, 0)
        plsc.subcore_barrier()

        def wb(kk, carry):
            r = s * rpt + kk * ZR
            pltpu.sync_copy(acc.at[pl.ds(r, ZR)], out.at[pl.ds(c * acc_rows + r, ZR)])
            return carry

        lax.fori_loop(0, rpt // ZR, wb, 0)

    return k


_pass1 = _seg_pass(144, ACC_E, gather_edge=False, col_split=False, tbl_rows=V_TBL, NB=2)
_pass2 = _seg_pass(72, ACC_V, gather_edge=True, col_split=True, tbl_rows=2 * E_TBL, NB=4)
_pass3 = _seg_pass(128, ACC_E, gather_edge=False, col_split=True, tbl_rows=2 * V_TBL, NB=4)
_pass4 = _seg_pass(48, ACC_V, gather_edge=True, col_split=False, tbl_rows=E_TBL, NB=4)


def _tc_call(body, out_shapes):
    return pl.pallas_call(body, out_shape=out_shapes)


def _row_mask(x):
    return jnp.where(lax.broadcasted_iota(jnp.int32, (E_TBL, 1), 0) < N_E, x, 0.0)


def _kb(p1, w1, b1, ae1, cnt_ref, t1a_ref, t1b_ref):
    S = p1[0:E_TBL, :] + p1[ACC_E:ACC_E + E_TBL, :]
    cnt = S[:, IN_C]
    cntc = jnp.maximum(cnt, 1.0)
    Xagg = S[:, :IN_C] / cntc[:, None]
    Yall = jnp.dot(Xagg, w1[...], preferred_element_type=jnp.float32) + b1[...][None, :]
    YEs, Es = [], []
    for h in range(H):
        Yh = Yall[:, h * HID:(h + 1) * HID]
        a = jnp.dot(Yh, ae1[h, :], preferred_element_type=jnp.float32)
        sle = jnp.where(a >= 0, a, NEG * a)
        E = jnp.exp(sle - jnp.max(sle))
        YEs.append(Yh * E[:, None])
        Es.append(E[:, None])
    z4 = jnp.zeros((E_TBL, 4), jnp.float32)
    q0 = _row_mask(jnp.concatenate([YEs[0]] + Es + [z4], axis=1))
    q1 = _row_mask(jnp.concatenate([YEs[1], z4, z4], axis=1))
    q2 = _row_mask(jnp.concatenate([YEs[2], z4, z4], axis=1))
    q3 = _row_mask(jnp.concatenate([YEs[3], z4, z4], axis=1))
    t1a_ref[...] = jnp.concatenate([q0, q1], axis=0)
    t1b_ref[...] = jnp.concatenate([q2, q3], axis=0)
    cnt_ref[...] = cnt


def _kd(p2a, p2b, xh_ref):
    qs = [p2a[0:V_TBL, :], p2a[ACC_V:ACC_V + V_TBL, :],
          p2b[0:V_TBL, :], p2b[ACC_V:ACC_V + V_TBL, :]]
    den = qs[0][:, HID:HID + H]
    R = 1.0 / (den + 1e-12)
    o = jnp.concatenate(
        [qs[h][:, :HID] * R[:, h:h + 1] for h in range(H)], axis=1)
    Xh = jnp.where(o > 0, o, jnp.exp(jnp.minimum(o, 0.0)) - 1.0)
    xh_ref[...] = jnp.concatenate([Xh[:, :IN_C], Xh[:, IN_C:]], axis=0)


def _ke(p3, cnt, w2, b2, ae2, t2_ref):
    S2 = jnp.concatenate([p3[0:E_TBL, :], p3[ACC_E:ACC_E + E_TBL, :]], axis=1)
    cntc = jnp.maximum(cnt[...], 1.0)
    X2 = S2 / cntc[:, None]
    Y2 = jnp.dot(X2, w2[...], preferred_element_type=jnp.float32) + b2[...][None, :]
    a = jnp.dot(Y2, ae2[...], preferred_element_type=jnp.float32)
    sle = jnp.where(a >= 0, a, NEG * a)
    E2 = jnp.exp(sle - jnp.max(sle))
    t2_ref[...] = _row_mask(jnp.concatenate(
        [Y2 * E2[:, None], E2[:, None], jnp.zeros((E_TBL, 7), jnp.float32)], axis=1))


def _kf(p4, out_ref):
    acc = p4[0:N_V, :] + p4[ACC_V:ACC_V + N_V, :]
    o = acc[:, :CLS] / (acc[:, CLS:CLS + 1] + 1e-12)
    out_ref[...] = jnp.where(o > 0, o, jnp.exp(jnp.minimum(o, 0.0)) - 1.0)


def kernel(X, pair_v, pair_e, W1, b1, ae1, W2, b2, ae2):
    f32 = jnp.float32
    npad = P_PAD - P
    pv = jnp.concatenate([pair_v, jnp.full((npad,), N_V, jnp.int32)])
    pe = jnp.concatenate([pair_e, jnp.full((npad,), N_E, jnp.int32)])
    pairs = jnp.left_shift(pe, 14) | pv
    X1t = jnp.concatenate(
        [jnp.concatenate([X, jnp.ones((N_V, 1), f32), jnp.zeros((N_V, 15), f32)], axis=1),
         jnp.zeros((V_TBL - N_V, 144), f32)], axis=0)
    W1cat = jnp.transpose(W1, (1, 0, 2)).reshape(IN_C, H * HID)
    b1cat = b1.reshape(H * HID)

    p1 = _pass1(X1t, pairs)
    cnt, T1a, T1b = _tc_call(
        _kb, (jax.ShapeDtypeStruct((E_TBL,), f32),
              jax.ShapeDtypeStruct((2 * E_TBL, 72), f32),
              jax.ShapeDtypeStruct((2 * E_TBL, 72), f32)))(p1, W1cat, b1cat, ae1)
    p2a = _pass2(T1a, pairs)
    p2b = _pass2(T1b, pairs)
    Xh2 = _tc_call(_kd, jax.ShapeDtypeStruct((2 * V_TBL, IN_C), f32))(p2a, p2b)
    p3 = _pass3(Xh2, pairs)
    T2 = _tc_call(_ke, jax.ShapeDtypeStruct((E_TBL, 48), f32))(p3, cnt, W2, b2, ae2)
    p4 = _pass4(T2, pairs)
    out = _tc_call(_kf, jax.ShapeDtypeStruct((N_V, CLS), f32))(p4)
    return out


# speedup vs baseline: 30.3906x; 1.2195x over previous
import functools

import jax
import jax.numpy as jnp
from jax import lax
from jax.experimental import pallas as pl
from jax.experimental.pallas import tpu as pltpu
from jax.experimental.pallas import tpu_sc as plsc

N_V = 10000
N_E = 5000
P = 320000
H = 4
IN_C = 128
HID = 64
CLS = 40
NEG = 0.2

NC, NS = 2, 16
CH = 128
P_PAD = 327680
V_TBL = 10008
E_TBL = 5008
ACC_E = 5024
ACC_V = 10048
ZR = 157


def _seg_pass(D, acc_rows, gather_edge, col_split, tbl_rows, NB):
    mesh = plsc.VectorSubcoreMesh(core_axis_name="c", subcore_axis_name="s")
    per_tile = P_PAD // NS if col_split else P_PAD // (NS * NC)
    n_chunks = per_tile // CH
    rpt = acc_rows // NS

    @functools.partial(
        pl.kernel,
        mesh=mesh,
        compiler_params=pltpu.CompilerParams(use_tc_tiling_on_sc=False),
        out_type=jax.ShapeDtypeStruct((NC * acc_rows, D), jnp.float32),
        scratch_types=(
            [pltpu.VMEM((CH,), jnp.int32)] * (3 * NB)
            + [pltpu.VMEM((CH, D), jnp.float32)] * NB
            + [pltpu.VMEM((ZR, D), jnp.float32),
               pltpu.VMEM_SHARED((acc_rows, D), jnp.float32)]
            + [pltpu.SemaphoreType.DMA] * NB
        ),
    )
    def k(tbl, pairs, out, *scr):
        idxps = scr[0:NB]
        idxgs = scr[NB:2 * NB]
        idxss = scr[2 * NB:3 * NB]
        rowss = scr[3 * NB:4 * NB]
        zbuf = scr[4 * NB]
        acc = scr[4 * NB + 1]
        sems = scr[4 * NB + 2:4 * NB + 2 + NB]
        c = lax.axis_index("c")
        s = lax.axis_index("s")

        def zrow(i, carry):
            for j in range(D // 16):
                zbuf[i, pl.ds(j * 16, 16)] = jnp.zeros((16,), jnp.float32)
            return carry

        lax.fori_loop(0, ZR, zrow, 0)

        def zacc(kk, carry):
            pltpu.sync_copy(zbuf, acc.at[pl.ds(s * rpt + kk * ZR, ZR)])
            return carry

        lax.fori_loop(0, rpt // ZR, zacc, 0)
        plsc.subcore_barrier()

        if col_split:
            start = s * per_tile
            goff = c * (tbl_rows // 2)
        else:
            start = (s * NC + c) * per_tile
            goff = 0

        def prefetch(kk, b):
            pltpu.sync_copy(pairs.at[pl.ds(start + kk * CH, CH)], idxps[b])
            for j in range(CH // 16):
                sl = pl.ds(j * 16, 16)
                v = idxps[b][sl]
                pe = lax.shift_right_logical(v, 14)
                pv = lax.bitwise_and(v, 16383)
                idxgs[b][sl] = (pe if gather_edge else pv) + goff
                idxss[b][sl] = pv if gather_edge else pe
            pltpu.async_copy(tbl.at[idxgs[b]], rowss[b], sems[b])

        def drain(b):
            pltpu.make_async_copy(tbl.at[idxgs[b]], rowss[b], sems[b]).wait()
            pltpu.sync_copy(rowss[b], acc.at[idxss[b]], add=True)

        for b in range(NB):
            prefetch(b, b)

        def step(i, carry):
            for b in range(NB):
                drain(b)
                nxt = NB * i + b + NB

                @pl.when(nxt < n_chunks)
                def _(nxt=nxt, b=b):
                    prefetch(nxt, b)

            return carry

        lax.fori_loop(0, n_chunks // NB, step, 0)
        plsc.subcore_barrier()

        def wb(kk, carry):
            r = s * rpt + kk * ZR
            pltpu.sync_copy(acc.at[pl.ds(r, ZR)], out.at[pl.ds(c * acc_rows + r, ZR)])
            return carry

        lax.fori_loop(0, rpt // ZR, wb, 0)

    return k


_pass1 = _seg_pass(144, ACC_E, gather_edge=False, col_split=False, tbl_rows=V_TBL, NB=2)
_pass2 = _seg_pass(72, ACC_V, gather_edge=True, col_split=True, tbl_rows=2 * E_TBL, NB=4)
_pass3 = _seg_pass(48, ACC_E, gather_edge=False, col_split=False, tbl_rows=V_TBL, NB=4)
_pass4 = _seg_pass(48, ACC_V, gather_edge=True, col_split=False, tbl_rows=E_TBL, NB=4)


def _tc_call(body, out_shapes):
    return pl.pallas_call(body, out_shape=out_shapes)


def _row_mask(x):
    return jnp.where(lax.broadcasted_iota(jnp.int32, (E_TBL, 1), 0) < N_E, x, 0.0)


def _kb(p1, w1, b1, ae1, cnt_ref, t1a_ref, t1b_ref):
    S = p1[0:E_TBL, :] + p1[ACC_E:ACC_E + E_TBL, :]
    cnt = S[:, IN_C]
    cntc = jnp.maximum(cnt, 1.0)
    Xagg = S[:, :IN_C] / cntc[:, None]
    Yall = jnp.dot(Xagg, w1[...], preferred_element_type=jnp.float32) + b1[...][None, :]
    YEs, Es = [], []
    for h in range(H):
        Yh = Yall[:, h * HID:(h + 1) * HID]
        a = jnp.dot(Yh, ae1[h, :], preferred_element_type=jnp.float32)
        sle = jnp.where(a >= 0, a, NEG * a)
        E = jnp.exp(sle - jnp.max(sle))
        YEs.append(Yh * E[:, None])
        Es.append(E[:, None])
    z4 = jnp.zeros((E_TBL, 4), jnp.float32)
    q0 = _row_mask(jnp.concatenate([YEs[0]] + Es + [z4], axis=1))
    q1 = _row_mask(jnp.concatenate([YEs[1], z4, z4], axis=1))
    q2 = _row_mask(jnp.concatenate([YEs[2], z4, z4], axis=1))
    q3 = _row_mask(jnp.concatenate([YEs[3], z4, z4], axis=1))
    t1a_ref[...] = jnp.concatenate([q0, q1], axis=0)
    t1b_ref[...] = jnp.concatenate([q2, q3], axis=0)
    cnt_ref[...] = cnt


def _kd(p2a, p2b, w2, z_ref):
    qs = [p2a[0:V_TBL, :], p2a[ACC_V:ACC_V + V_TBL, :],
          p2b[0:V_TBL, :], p2b[ACC_V:ACC_V + V_TBL, :]]
    den = qs[0][:, HID:HID + H]
    R = 1.0 / (den + 1e-12)
    o = jnp.concatenate(
        [qs[h][:, :HID] * R[:, h:h + 1] for h in range(H)], axis=1)
    Xh = jnp.where(o > 0, o, jnp.exp(jnp.minimum(o, 0.0)) - 1.0)
    Z = jnp.dot(Xh, w2[...], preferred_element_type=jnp.float32)
    z_ref[...] = jnp.concatenate([Z, jnp.zeros((V_TBL, 8), jnp.float32)], axis=1)


def _ke(p3, cnt, b2, ae2, t2_ref):
    S2z = p3[0:E_TBL, :] + p3[ACC_E:ACC_E + E_TBL, :]
    cntc = jnp.maximum(cnt[...], 1.0)
    Y2 = S2z[:, :CLS] / cntc[:, None] + b2[...][None, :]
    a = jnp.dot(Y2, ae2[...], preferred_element_type=jnp.float32)
    sle = jnp.where(a >= 0, a, NEG * a)
    E2 = jnp.exp(sle - jnp.max(sle))
    t2_ref[...] = _row_mask(jnp.concatenate(
        [Y2 * E2[:, None], E2[:, None], jnp.zeros((E_TBL, 7), jnp.float32)], axis=1))


def _kf(p4, out_ref):
    acc = p4[0:N_V, :] + p4[ACC_V:ACC_V + N_V, :]
    o = acc[:, :CLS] / (acc[:, CLS:CLS + 1] + 1e-12)
    out_ref[...] = jnp.where(o > 0, o, jnp.exp(jnp.minimum(o, 0.0)) - 1.0)


def kernel(X, pair_v, pair_e, W1, b1, ae1, W2, b2, ae2):
    f32 = jnp.float32
    npad = P_PAD - P
    pv = jnp.concatenate([pair_v, jnp.full((npad,), N_V, jnp.int32)])
    pe = jnp.concatenate([pair_e, jnp.full((npad,), N_E, jnp.int32)])
    pairs = jnp.left_shift(pe, 14) | pv
    X1t = jnp.concatenate(
        [jnp.concatenate([X, jnp.ones((N_V, 1), f32), jnp.zeros((N_V, 15), f32)], axis=1),
         jnp.zeros((V_TBL - N_V, 144), f32)], axis=0)
    W1cat = jnp.transpose(W1, (1, 0, 2)).reshape(IN_C, H * HID)
    b1cat = b1.reshape(H * HID)

    p1 = _pass1(X1t, pairs)
    cnt, T1a, T1b = _tc_call(
        _kb, (jax.ShapeDtypeStruct((E_TBL,), f32),
              jax.ShapeDtypeStruct((2 * E_TBL, 72), f32),
              jax.ShapeDtypeStruct((2 * E_TBL, 72), f32)))(p1, W1cat, b1cat, ae1)
    p2a = _pass2(T1a, pairs)
    p2b = _pass2(T1b, pairs)
    Z = _tc_call(_kd, jax.ShapeDtypeStruct((V_TBL, 48), f32))(p2a, p2b, W2)
    p3 = _pass3(Z, pairs)
    T2 = _tc_call(_ke, jax.ShapeDtypeStruct((E_TBL, 48), f32))(p3, cnt, b2, ae2)
    p4 = _pass4(T2, pairs)
    out = _tc_call(_kf, jax.ShapeDtypeStruct((N_V, CLS), f32))(p4)
    return out


# speedup vs baseline: 32.2634x; 1.0616x over previous
import functools

import jax
import jax.numpy as jnp
from jax import lax
from jax.experimental import pallas as pl
from jax.experimental.pallas import tpu as pltpu
from jax.experimental.pallas import tpu_sc as plsc

N_V = 10000
N_E = 5000
P = 320000
H = 4
IN_C = 128
HID = 64
CLS = 40
NEG = 0.2

NC, NS = 2, 16
CH = 128
P_PAD = 327680
V_TBL = 10008
E_TBL = 5008
ACC_E = 5024
ACC_V = 10048
ZR = 157


def _seg_pass(D, acc_rows, gather_edge, col_split, tbl_rows, NB, n_tbl=1):
    mesh = plsc.VectorSubcoreMesh(core_axis_name="c", subcore_axis_name="s")
    per_tile = P_PAD // NS if col_split else P_PAD // (NS * NC)
    n_chunks = per_tile // CH
    rpt = acc_rows // NS

    @functools.partial(
        pl.kernel,
        mesh=mesh,
        compiler_params=pltpu.CompilerParams(use_tc_tiling_on_sc=False),
        out_type=jax.ShapeDtypeStruct((n_tbl * NC * acc_rows, D), jnp.float32),
        scratch_types=(
            [pltpu.VMEM((CH,), jnp.int32)] * (3 * NB)
            + [pltpu.VMEM((CH, D), jnp.float32)] * NB
            + [pltpu.VMEM((ZR, D), jnp.float32),
               pltpu.VMEM_SHARED((acc_rows, D), jnp.float32)]
            + [pltpu.SemaphoreType.DMA] * NB
        ),
    )
    def k(*args):
        tbls = args[0:n_tbl]
        pairs = args[n_tbl]
        out = args[n_tbl + 1]
        scr = args[n_tbl + 2:]
        idxps = scr[0:NB]
        idxgs = scr[NB:2 * NB]
        idxss = scr[2 * NB:3 * NB]
        rowss = scr[3 * NB:4 * NB]
        zbuf = scr[4 * NB]
        acc = scr[4 * NB + 1]
        sems = scr[4 * NB + 2:4 * NB + 2 + NB]
        c = lax.axis_index("c")
        s = lax.axis_index("s")

        def zrow(i, carry):
            for j in range(D // 16):
                zbuf[i, pl.ds(j * 16, 16)] = jnp.zeros((16,), jnp.float32)
            return carry

        lax.fori_loop(0, ZR, zrow, 0)

        if col_split:
            start = s * per_tile
            goff = c * (tbl_rows // 2)
        else:
            start = (s * NC + c) * per_tile
            goff = 0

        for phase in range(n_tbl):
            tbl = tbls[phase]

            def zacc(kk, carry):
                pltpu.sync_copy(zbuf, acc.at[pl.ds(s * rpt + kk * ZR, ZR)])
                return carry

            lax.fori_loop(0, rpt // ZR, zacc, 0)
            plsc.subcore_barrier()

            def prefetch(kk, b, tbl=tbl):
                pltpu.sync_copy(pairs.at[pl.ds(start + kk * CH, CH)], idxps[b])
                for j in range(CH // 16):
                    sl = pl.ds(j * 16, 16)
                    v = idxps[b][sl]
                    pe = lax.shift_right_logical(v, 14)
                    pv = lax.bitwise_and(v, 16383)
                    idxgs[b][sl] = (pe if gather_edge else pv) + goff
                    idxss[b][sl] = pv if gather_edge else pe
                pltpu.async_copy(tbl.at[idxgs[b]], rowss[b], sems[b])

            def drain(b, tbl=tbl):
                pltpu.make_async_copy(tbl.at[idxgs[b]], rowss[b], sems[b]).wait()
                pltpu.sync_copy(rowss[b], acc.at[idxss[b]], add=True)

            for b in range(NB):
                prefetch(b, b)

            def step(i, carry, prefetch=prefetch, drain=drain):
                for b in range(NB):
                    drain(b)
                    nxt = NB * i + b + NB

                    @pl.when(nxt < n_chunks)
                    def _(nxt=nxt, b=b):
                        prefetch(nxt, b)

                return carry

            lax.fori_loop(0, n_chunks // NB, step, 0)
            plsc.subcore_barrier()

            obase = (phase * NC + c) * acc_rows

            def wb(kk, carry, obase=obase):
                r = s * rpt + kk * ZR
                pltpu.sync_copy(acc.at[pl.ds(r, ZR)], out.at[pl.ds(obase + r, ZR)])
                return carry

            lax.fori_loop(0, rpt // ZR, wb, 0)

    return k


_pass1 = _seg_pass(144, ACC_E, gather_edge=False, col_split=False, tbl_rows=V_TBL, NB=2)
_pass2 = _seg_pass(72, ACC_V, gather_edge=True, col_split=True, tbl_rows=2 * E_TBL,
                   NB=4, n_tbl=2)
_pass3 = _seg_pass(48, ACC_E, gather_edge=False, col_split=False, tbl_rows=V_TBL, NB=4)
_pass4 = _seg_pass(48, ACC_V, gather_edge=True, col_split=False, tbl_rows=E_TBL, NB=4)


def _tc_call(body, out_shapes):
    return pl.pallas_call(body, out_shape=out_shapes)


def _row_mask(x):
    return jnp.where(lax.broadcasted_iota(jnp.int32, (E_TBL, 1), 0) < N_E, x, 0.0)


def _kb(p1, w1, b1, ae1, cnt_ref, t1a_ref, t1b_ref):
    S = p1[0:E_TBL, :] + p1[ACC_E:ACC_E + E_TBL, :]
    cnt = S[:, IN_C]
    cntc = jnp.maximum(cnt, 1.0)
    Xagg = S[:, :IN_C] / cntc[:, None]
    Yall = jnp.dot(Xagg, w1[...], preferred_element_type=jnp.float32) + b1[...][None, :]
    YEs, Es = [], []
    for h in range(H):
        Yh = Yall[:, h * HID:(h + 1) * HID]
        a = jnp.dot(Yh, ae1[h, :], preferred_element_type=jnp.float32)
        sle = jnp.where(a >= 0, a, NEG * a)
        E = jnp.exp(sle - jnp.max(sle))
        YEs.append(Yh * E[:, None])
        Es.append(E[:, None])
    z4 = jnp.zeros((E_TBL, 4), jnp.float32)
    q0 = _row_mask(jnp.concatenate([YEs[0]] + Es + [z4], axis=1))
    q1 = _row_mask(jnp.concatenate([YEs[1], z4, z4], axis=1))
    q2 = _row_mask(jnp.concatenate([YEs[2], z4, z4], axis=1))
    q3 = _row_mask(jnp.concatenate([YEs[3], z4, z4], axis=1))
    t1a_ref[...] = jnp.concatenate([q0, q1], axis=0)
    t1b_ref[...] = jnp.concatenate([q2, q3], axis=0)
    cnt_ref[...] = cnt


def _kd(p2, w2, z_ref):
    qs = [p2[i * ACC_V:i * ACC_V + V_TBL, :] for i in range(4)]
    den = qs[0][:, HID:HID + H]
    R = 1.0 / (den + 1e-12)
    o = jnp.concatenate(
        [qs[h][:, :HID] * R[:, h:h + 1] for h in range(H)], axis=1)
    Xh = jnp.where(o > 0, o, jnp.exp(jnp.minimum(o, 0.0)) - 1.0)
    Z = jnp.dot(Xh, w2[...], preferred_element_type=jnp.float32)
    z_ref[...] = jnp.concatenate([Z, jnp.zeros((V_TBL, 8), jnp.float32)], axis=1)


def _ke(p3, cnt, b2, ae2, t2_ref):
    S2z = p3[0:E_TBL, :] + p3[ACC_E:ACC_E + E_TBL, :]
    cntc = jnp.maximum(cnt[...], 1.0)
    Y2 = S2z[:, :CLS] / cntc[:, None] + b2[...][None, :]
    a = jnp.dot(Y2, ae2[...], preferred_element_type=jnp.float32)
    sle = jnp.where(a >= 0, a, NEG * a)
    E2 = jnp.exp(sle - jnp.max(sle))
    t2_ref[...] = _row_mask(jnp.concatenate(
        [Y2 * E2[:, None], E2[:, None], jnp.zeros((E_TBL, 7), jnp.float32)], axis=1))


def _kf(p4, out_ref):
    acc = p4[0:N_V, :] + p4[ACC_V:ACC_V + N_V, :]
    o = acc[:, :CLS] / (acc[:, CLS:CLS + 1] + 1e-12)
    out_ref[...] = jnp.where(o > 0, o, jnp.exp(jnp.minimum(o, 0.0)) - 1.0)


def kernel(X, pair_v, pair_e, W1, b1, ae1, W2, b2, ae2):
    f32 = jnp.float32
    npad = P_PAD - P
    pv = jnp.concatenate([pair_v, jnp.full((npad,), N_V, jnp.int32)])
    pe = jnp.concatenate([pair_e, jnp.full((npad,), N_E, jnp.int32)])
    pairs = jnp.left_shift(pe, 14) | pv
    X1t = jnp.concatenate(
        [jnp.concatenate([X, jnp.ones((N_V, 1), f32), jnp.zeros((N_V, 15), f32)], axis=1),
         jnp.zeros((V_TBL - N_V, 144), f32)], axis=0)
    W1cat = jnp.transpose(W1, (1, 0, 2)).reshape(IN_C, H * HID)
    b1cat = b1.reshape(H * HID)

    p1 = _pass1(X1t, pairs)
    cnt, T1a, T1b = _tc_call(
        _kb, (jax.ShapeDtypeStruct((E_TBL,), f32),
              jax.ShapeDtypeStruct((2 * E_TBL, 72), f32),
              jax.ShapeDtypeStruct((2 * E_TBL, 72), f32)))(p1, W1cat, b1cat, ae1)
    p2 = _pass2(T1a, T1b, pairs)
    Z = _tc_call(_kd, jax.ShapeDtypeStruct((V_TBL, 48), f32))(p2, W2)
    p3 = _pass3(Z, pairs)
    T2 = _tc_call(_ke, jax.ShapeDtypeStruct((E_TBL, 48), f32))(p3, cnt, b2, ae2)
    p4 = _pass4(T2, pairs)
    out = _tc_call(_kf, jax.ShapeDtypeStruct((N_V, CLS), f32))(p4)
    return out
